# baseline (speedup 1.0000x reference)
import functools

import jax
import jax.numpy as jnp
from jax import lax
from jax.experimental import pallas as pl
from jax.experimental.pallas import tpu as pltpu
from jax.experimental.pallas import tpu_sc as plsc

BATCH = 128
SEQ = 200
D = 2048
TOTAL = BATCH * SEQ
NC = 2
NS = 16
NW = NC * NS
B_PER_W = TOTAL // NW
CHUNK = 16
NCHUNKS = B_PER_W // CHUNK
NB = 3


def _body(idx_hbm, table_hbm, out_hbm, idx_v, rows_v, gsem, wsem):
    wid = lax.axis_index("s") * NC + lax.axis_index("c")
    base = wid * B_PER_W
    pltpu.sync_copy(idx_hbm.at[wid], idx_v)

    def g_copy(j, b):
        return pltpu.make_async_copy(table_hbm.at[idx_v.at[j]], rows_v.at[b], gsem)

    def w_copy(j, b):
        return pltpu.make_async_copy(
            rows_v.at[b], out_hbm.at[pl.ds(base + j * CHUNK, CHUNK)], wsem)

    def step(j, b, jw, jg):
        g_copy(j, b).wait()
        w_copy(j, b).start()
        if jw is not None:
            w_copy(jw, (b + 1) % NB).wait()
        if jg is not None:
            g_copy(jg, (b + 1) % NB).start()

    g_copy(0, 0).start()
    step(0, 0, None, 1)
    step(1, 1, None, 2)
    step(2, 2, 0, 3)

    def group(gi, carry):
        j0 = 3 * gi + 3
        for t in range(NB):
            step(j0 + t, t, j0 + t - 2, j0 + t + 1)
        return carry

    lax.fori_loop(0, (NCHUNKS - 5) // NB, group, 0)

    step(NCHUNKS - 2, (NCHUNKS - 2) % NB, NCHUNKS - 4, NCHUNKS - 1)
    step(NCHUNKS - 1, (NCHUNKS - 1) % NB, NCHUNKS - 3, None)
    w_copy(NCHUNKS - 2, (NCHUNKS - 2) % NB).wait()
    w_copy(NCHUNKS - 1, (NCHUNKS - 1) % NB).wait()


_gather = functools.partial(
    pl.kernel,
    mesh=plsc.VectorSubcoreMesh(core_axis_name="c", subcore_axis_name="s"),
    out_type=jax.ShapeDtypeStruct((TOTAL, D), jnp.float32),
    scratch_types=[
        pltpu.VMEM((NCHUNKS, CHUNK), jnp.int32),
        pltpu.VMEM((NB, CHUNK, D), jnp.float32),
        pltpu.SemaphoreType.DMA,
        pltpu.SemaphoreType.DMA,
    ],
)(_body)


def kernel(indices, weight):
    idx = indices.astype(jnp.int32).reshape(NW, NCHUNKS, CHUNK)
    out = _gather(idx, weight)
    return out.reshape(BATCH, SEQ, D)
